# baseline (speedup 1.0000x reference)
import functools

import jax
import jax.numpy as jnp
from jax.experimental import pallas as pl
from jax.experimental.pallas import tpu as pltpu

_TOP_K = 5


def _top5_lanes(v):
    iota = jax.lax.broadcasted_iota(jnp.int32, v.shape, 2)
    x = v
    ms = []
    for _ in range(_TOP_K):
        m = jnp.max(x, axis=2, keepdims=True)
        ms.append(m)
        idx = jnp.min(jnp.where(x == m, iota, 2**30), axis=2, keepdims=True)
        x = jnp.where(iota == idx, -jnp.inf, x)
    return ms


def _lanes5(ms):
    li = jax.lax.broadcasted_iota(jnp.int32, (1, 1, _TOP_K), 2)
    out = jnp.zeros((1, 1, _TOP_K), jnp.float32)
    for i in range(_TOP_K):
        out = jnp.where(li == i, ms[i], out)
    return out


def _fused_kernel(inv_hw, x_ref, d_ref, o_ref, ksm_ref, klab_ref, kalt_ref):
    C = x_ref.shape[1]
    x = x_ref[0]
    e = jnp.exp(x)
    tot = jnp.sum(e, axis=0)
    recip = 1.0 / tot
    sd = jnp.sum(e * (recip * inv_hw), axis=(1, 2), keepdims=True)

    sdl = sd.reshape(1, 1, C)
    dist = d_ref[...]
    ms = _top5_lanes(sdl)
    thresh = ms[-1]
    mask = (sdl > thresh).astype(jnp.float32)
    rl = (dist - sdl) * mask
    r = rl.reshape(C, 1, 1)

    o_ref[0] = e * recip + r

    sum_a = jnp.sum(sdl * mask, axis=2, keepdims=True)
    top_a = _lanes5([jnp.where(m > thresh, m, 0.0) for m in ms])
    ksm_ref[...] = top_a / sum_a

    b = dist * mask
    sum_b = jnp.sum(b, axis=2, keepdims=True)
    top_b = _lanes5(_top5_lanes(b))
    klab_ref[...] = top_b / (sum_b + 1e-12)
    kalt_ref[...] = top_b / sum_b


def kernel(logits, distribution):
    N, C, H, W = logits.shape

    dist_l = distribution.reshape(N, 1, C)
    corrected, ksm, klab, kalt = pl.pallas_call(
        functools.partial(_fused_kernel, 1.0 / (H * W)),
        grid=(N,),
        in_specs=[
            pl.BlockSpec((1, C, H, W), lambda n: (n, 0, 0, 0)),
            pl.BlockSpec((1, 1, C), lambda n: (n, 0, 0)),
        ],
        out_specs=[
            pl.BlockSpec((1, C, H, W), lambda n: (n, 0, 0, 0)),
            pl.BlockSpec((1, 1, _TOP_K), lambda n: (n, 0, 0)),
            pl.BlockSpec((1, 1, _TOP_K), lambda n: (n, 0, 0)),
            pl.BlockSpec((1, 1, _TOP_K), lambda n: (n, 0, 0)),
        ],
        out_shape=[
            jax.ShapeDtypeStruct((N, C, H, W), jnp.float32),
            jax.ShapeDtypeStruct((N, 1, _TOP_K), jnp.float32),
            jax.ShapeDtypeStruct((N, 1, _TOP_K), jnp.float32),
            jax.ShapeDtypeStruct((N, 1, _TOP_K), jnp.float32),
        ],
        compiler_params=pltpu.CompilerParams(
            dimension_semantics=("parallel",)),
    )(logits, dist_l)

    k1 = klab.reshape(N, _TOP_K, 1, 1)
    k2 = ksm.reshape(N, _TOP_K, 1, 1)
    k3 = kalt.reshape(N, _TOP_K, 1, 1)
    return (corrected, k1, k2, k3)


# speedup vs baseline: 1.0935x; 1.0935x over previous
import functools

import jax
import jax.numpy as jnp
from jax.experimental import pallas as pl
from jax.experimental.pallas import tpu as pltpu

_TOP_K = 5


def _top5_lanes(v, axis):
    iota = jax.lax.broadcasted_iota(jnp.int32, v.shape, axis)
    x = v
    ms = []
    for _ in range(_TOP_K):
        m = jnp.max(x, axis=axis, keepdims=True)
        ms.append(m)
        idx = jnp.min(jnp.where(x == m, iota, 2**30), axis=axis,
                      keepdims=True)
        x = jnp.where(iota == idx, -jnp.inf, x)
    return ms


def _fused_kernel(inv_hw, x_ref, d_ref, o_ref, sd_ref):
    C = x_ref.shape[1]
    x = x_ref[0]
    e = jnp.exp(x)
    tot = jnp.sum(e, axis=0)
    recip = 1.0 / tot
    sd = jnp.sum(e * recip, axis=(1, 2), keepdims=True) * inv_hw

    sdl = sd.reshape(1, 1, C)
    sd_ref[...] = sdl
    dist = d_ref[...]
    thresh = _top5_lanes(sdl, 2)[-1]
    mask = (sdl > thresh).astype(jnp.float32)
    rl = (dist - sdl) * mask
    r = rl.reshape(C, 1, 1)

    o_ref[0] = e * recip + r


def _topk_kernel(sd_ref, d_ref, ksm_ref, klab_ref, kalt_ref):
    sd = sd_ref[:, 0, :]
    dist = d_ref[:, 0, :]
    ms = _top5_lanes(sd, 1)
    thresh = ms[-1]
    mask = (sd > thresh).astype(jnp.float32)

    li = jax.lax.broadcasted_iota(jnp.int32, (sd.shape[0], _TOP_K), 1)

    def lanes5(vals):
        out = jnp.zeros((sd.shape[0], _TOP_K), jnp.float32)
        for i in range(_TOP_K):
            out = jnp.where(li == i, vals[i], out)
        return out

    sum_a = jnp.sum(jnp.where(mask > 0, sd, 0.0), axis=1, keepdims=True)
    top_a = lanes5([jnp.where(m > thresh, m, 0.0) for m in ms])
    ksm_ref[...] = top_a / sum_a

    b = jnp.where(mask > 0, dist, 0.0)
    sum_b = jnp.sum(b, axis=1, keepdims=True)
    top_b = lanes5(_top5_lanes(b, 1))
    klab_ref[...] = top_b / (sum_b + 1e-12)
    kalt_ref[...] = top_b / sum_b


def kernel(logits, distribution):
    N, C, H, W = logits.shape

    dist_l = distribution.reshape(N, 1, C)
    corrected, sd = pl.pallas_call(
        functools.partial(_fused_kernel, 1.0 / (H * W)),
        grid=(N,),
        in_specs=[
            pl.BlockSpec((1, C, H, W), lambda n: (n, 0, 0, 0)),
            pl.BlockSpec((1, 1, C), lambda n: (n, 0, 0)),
        ],
        out_specs=[
            pl.BlockSpec((1, C, H, W), lambda n: (n, 0, 0, 0)),
            pl.BlockSpec((1, 1, C), lambda n: (n, 0, 0)),
        ],
        out_shape=[
            jax.ShapeDtypeStruct((N, C, H, W), jnp.float32),
            jax.ShapeDtypeStruct((N, 1, C), jnp.float32),
        ],
        compiler_params=pltpu.CompilerParams(
            dimension_semantics=("parallel",)),
    )(logits, dist_l)

    ksm, klab, kalt = pl.pallas_call(
        _topk_kernel,
        in_specs=[
            pl.BlockSpec((N, 1, C), lambda: (0, 0, 0)),
            pl.BlockSpec((N, 1, C), lambda: (0, 0, 0)),
        ],
        out_specs=[
            pl.BlockSpec((N, _TOP_K), lambda: (0, 0)),
            pl.BlockSpec((N, _TOP_K), lambda: (0, 0)),
            pl.BlockSpec((N, _TOP_K), lambda: (0, 0)),
        ],
        out_shape=[
            jax.ShapeDtypeStruct((N, _TOP_K), jnp.float32),
            jax.ShapeDtypeStruct((N, _TOP_K), jnp.float32),
            jax.ShapeDtypeStruct((N, _TOP_K), jnp.float32),
        ],
    )(sd, dist_l)

    k1 = klab.reshape(N, _TOP_K, 1, 1)
    k2 = ksm.reshape(N, _TOP_K, 1, 1)
    k3 = kalt.reshape(N, _TOP_K, 1, 1)
    return (corrected, k1, k2, k3)
